# Initial kernel scaffold; baseline (speedup 1.0000x reference)
import jax
import jax.numpy as jnp
from jax.experimental import pallas as pl


def kernel(heatmaps, center, cameraMatrices):
    raise NotImplementedError("write your pallas kernel here")


# speedup vs baseline: 4.7046x; 4.7046x over previous
import functools

import jax
import jax.numpy as jnp
from jax import lax
from jax.experimental import pallas as pl
from jax.experimental.pallas import tpu as pltpu
from jax.experimental.pallas import tpu_sc as plsc

GRID = 64
G3 = GRID ** 3
IMG_W, IMG_H = 640, 512
WH, HH = IMG_W // 2, IMG_H // 2
HW = WH * HH
B, C, J = 2, 8, 23
JP = 32
ROWS = B * G3
NBLK = ROWS // 128

NCORE, NSUBC = 2, 16
NW = NCORE * NSUBC
RPW = ROWS // NW
CHUNK = 4096
NCHUNK = RPW // CHUNK
NSUBCH = CHUNK // 128


def _idx_body(center_ref, cam_ref, out_ref):
    cam = pl.program_id(0)
    b = pl.program_id(1)
    n = pl.program_id(2)
    r = lax.broadcasted_iota(jnp.int32, (1, NSUBCH, 128), 1)
    q = lax.broadcasted_iota(jnp.int32, (1, NSUBCH, 128), 2)
    g = n * CHUNK + r * 128 + q
    x = g >> 12
    y = (g >> 6) & (GRID - 1)
    z = g & (GRID - 1)
    fx = (x - GRID // 2).astype(jnp.float32) * 2.0 + center_ref[b, 0]
    fy = (y - GRID // 2).astype(jnp.float32) * 2.0 + center_ref[b, 1]
    fz = (z - GRID // 2).astype(jnp.float32) * 2.0 + center_ref[b, 2]
    def _bf(t):
        return t.astype(jnp.bfloat16).astype(jnp.float32)

    fx, fy, fz = _bf(fx), _bf(fy), _bf(fz)
    m = [[_bf(cam_ref[b, cam, k, col]) for col in range(3)] for k in range(4)]
    xp = fx * m[0][0] + fy * m[1][0] + fz * m[2][0] + m[3][0]
    yp = fx * m[0][1] + fy * m[1][1] + fz * m[2][1] + m[3][1]
    zp = fx * m[0][2] + fy * m[1][2] + fz * m[2][2] + m[3][2]
    u = jnp.clip(xp / zp, 0.0, float(IMG_W - 1))
    v = jnp.clip(yp / zp, 0.0, float(IMG_H - 1))
    pix = (v * 0.5).astype(jnp.int32) * WH + (u * 0.5).astype(jnp.int32)
    out_ref[...] = pix + (b * C + cam) * HW


def _compute_idx(center, cameraMatrices):
    return pl.pallas_call(
        _idx_body,
        grid=(C, B, G3 // CHUNK),
        in_specs=[
            pl.BlockSpec(memory_space=pltpu.SMEM),
            pl.BlockSpec(memory_space=pltpu.SMEM),
        ],
        out_specs=pl.BlockSpec((1, NSUBCH, 128),
                               lambda cam, b, n: (cam, b * (G3 // CHUNK) + n, 0)),
        out_shape=jax.ShapeDtypeStruct((C, NBLK, 128), jnp.int32),
    )(center, cameraMatrices)


@functools.lru_cache(maxsize=1)
def _make_sc_gather():
    mesh = plsc.VectorSubcoreMesh(core_axis_name="c", subcore_axis_name="s")
    return functools.partial(
        pl.kernel,
        mesh=mesh,
        compiler_params=pltpu.CompilerParams(use_tc_tiling_on_sc=False),
        out_type=jax.ShapeDtypeStruct((ROWS, JP), jnp.float32),
        scratch_types=[
            pltpu.VMEM((C, NSUBCH, 128), jnp.int32),
            pltpu.VMEM((C, 128, JP), jnp.float32),
            pltpu.SemaphoreType.DMA,
        ],
    )(_sc_gather_body)


def _sc_gather_body(table_hbm, idx_hbm, out_hbm, idx_v, bufs, sem):
    wid = lax.axis_index("s") * NCORE + lax.axis_index("c")

    def chunk_body(k, carry):
        base = pl.multiple_of(wid * RPW + k * CHUNK, 128)
        bb = pl.multiple_of(base // 128, 32)
        for cam in range(C):
            pltpu.sync_copy(idx_hbm.at[cam, pl.ds(bb, NSUBCH)], idx_v.at[cam])

        def sub_body(jsub, carry2):
            cps = []
            for cam in range(C):
                cps.append(pltpu.async_copy(
                    table_hbm.at[idx_v.at[cam, jsub]], bufs.at[cam], sem))
            for cp in cps:
                cp.wait()

            def acc_body(rrow, carry3):
                for half in range(JP // 16):
                    sl = pl.ds(half * 16, 16)
                    s = bufs[0, rrow, sl]
                    for cam in range(1, C):
                        s = s + bufs[cam, rrow, sl]
                    bufs[0, rrow, sl] = s * (1.0 / C)
                return carry3

            lax.fori_loop(0, 128, acc_body, 0)
            pltpu.sync_copy(
                bufs.at[0],
                out_hbm.at[pl.ds(pl.multiple_of(base + jsub * 128, 128), 128)])
            return carry2

        lax.fori_loop(0, NSUBCH, sub_body, 0)
        return carry

    lax.fori_loop(0, NCHUNK, chunk_body, 0)


def kernel(heatmaps, center, cameraMatrices):
    t = jnp.transpose(heatmaps, (0, 1, 3, 4, 2))
    t = jnp.pad(t, ((0, 0), (0, 0), (0, 0), (0, 0), (0, JP - J)))
    table = t.reshape(B * C * HW, JP)
    idx = _compute_idx(center, cameraMatrices)
    out32 = _make_sc_gather()(table, idx)
    out = jnp.moveaxis(out32.reshape(B, G3, JP), 2, 1)[:, :J, :]
    return out.reshape(B, J, GRID, GRID, GRID)
